# baseline (speedup 1.0000x reference)
import functools

import jax
import jax.numpy as jnp
from jax import lax
from jax.experimental import pallas as pl
from jax.experimental.pallas import tpu as pltpu
from jax.experimental.pallas import tpu_sc as plsc

N = 10000
E = 320000
D = 128
L = 16
NC, NS = 2, 16
NW = NC * NS
NP = 10112

E_PER_W = E // NW
CHUNK = 80
N_CHUNKS = E_PER_W // CHUNK
ROWS_PER_TILE = NP // NS
DEPTH = 3


def _spmm_body(src_hbm, dst_hbm, vals_hbm, feat_hbm, out_hbm,
               src_all, dst0, dst1, dst2, vals0, vals1, vals2,
               rows0, rows1, rows2,
               dsem0, dsem1, dsem2, vsem0, vsem1, vsem2,
               gsem0, gsem1, gsem2, ssem0, ssem1, ssem2, acc_sh):
    cid = lax.axis_index("c")
    sid = lax.axis_index("s")
    wid = sid * NC + cid

    zvec = jnp.zeros((L,), jnp.float32)

    def zblk(e, _):
        for k in range(D // L):
            rows0[e, pl.ds(k * L, L)] = zvec
        return 0

    lax.fori_loop(0, CHUNK, zblk, 0)
    row0 = sid * ROWS_PER_TILE
    for i in range(7):
        pltpu.sync_copy(rows0, acc_sh.at[pl.ds(row0 + i * CHUNK, CHUNK)])
    pltpu.sync_copy(rows0.at[pl.ds(0, ROWS_PER_TILE - 7 * CHUNK)],
                    acc_sh.at[pl.ds(row0 + 7 * CHUNK,
                                    ROWS_PER_TILE - 7 * CHUNK)])
    e0 = wid * E_PER_W
    pltpu.sync_copy(src_hbm.at[pl.ds(e0, E_PER_W)], src_all)
    plsc.subcore_barrier()

    rows = (rows0, rows1, rows2)
    dst = (dst0, dst1, dst2)
    vals = (vals0, vals1, vals2)
    dsem = (dsem0, dsem1, dsem2)
    vsem = (vsem0, vsem1, vsem2)
    gsem = (gsem0, gsem1, gsem2)
    ssem = (ssem0, ssem1, ssem2)

    def scale(s, buf):
        def blk(b, _):
            ve = vals[s][pl.ds(b * L, L)]
            for j in range(L):
                e = b * L + j
                vj = jnp.full((L,), ve[j], dtype=jnp.float32)
                for k in range(D // L):
                    buf[e, pl.ds(k * L, L)] = buf[e, pl.ds(k * L, L)] * vj
            return 0
        lax.fori_loop(0, CHUNK // L, blk, 0)

    def issue_chunk(g, s):
        pltpu.async_copy(dst_hbm.at[pl.ds(e0 + g * CHUNK, CHUNK)],
                         dst[s], dsem[s])
        pltpu.async_copy(vals_hbm.at[pl.ds(e0 + g * CHUNK, CHUNK)],
                         vals[s], vsem[s])
        half = CHUNK // 2
        pltpu.async_copy(feat_hbm.at[src_all.at[pl.ds(g * CHUNK, half)]],
                         rows[s].at[pl.ds(0, half)], gsem[s])
        pltpu.async_copy(
            feat_hbm.at[src_all.at[pl.ds(g * CHUNK + half, half)]],
            rows[s].at[pl.ds(half, half)], gsem[s])

    issue_chunk(0, 0)
    issue_chunk(1, 1)

    def step(g, s, s2, last):
        pltpu.make_async_copy(feat_hbm.at[src_all.at[pl.ds(0, CHUNK)]],
                              rows[s], gsem[s]).wait()

        @pl.when(g >= 1)
        def _():
            pltpu.make_async_copy(rows[s2], acc_sh.at[dst[s2]],
                                  ssem[s2]).wait()

        if not last:
            @pl.when(g + 2 <= N_CHUNKS - 1)
            def _():
                issue_chunk(g + 2, s2)

        pltpu.make_async_copy(vals_hbm.at[pl.ds(e0, CHUNK)], vals[s],
                              vsem[s]).wait()
        scale(s, rows[s])
        pltpu.make_async_copy(dst_hbm.at[pl.ds(e0, CHUNK)], dst[s],
                              dsem[s]).wait()
        if last:
            pltpu.sync_copy(rows[s], acc_sh.at[dst[s]], add=True)
        else:
            pltpu.async_copy(rows[s], acc_sh.at[dst[s]], ssem[s], add=True)

    def outer(i, _):
        for b in range(DEPTH):
            step(i * DEPTH + b, b, (b + 2) % DEPTH, False)
        return 0

    lax.fori_loop(0, (N_CHUNKS - 2) // DEPTH, outer, 0)
    step(N_CHUNKS - 2, 0, 2, False)
    step(N_CHUNKS - 1, 1, 0, True)

    plsc.subcore_barrier()
    pltpu.sync_copy(acc_sh.at[pl.ds(row0, ROWS_PER_TILE)],
                    out_hbm.at[cid, pl.ds(row0, ROWS_PER_TILE)])


_spmm = functools.partial(
    pl.kernel,
    out_type=jax.ShapeDtypeStruct((NC, NP, D), jnp.float32),
    mesh=plsc.VectorSubcoreMesh(core_axis_name="c", subcore_axis_name="s",
                                num_cores=NC, num_subcores=NS),
    scratch_types=(
        [pltpu.VMEM((E_PER_W,), jnp.int32)]
        + [pltpu.VMEM((CHUNK,), jnp.int32) for _ in range(3)]
        + [pltpu.VMEM((CHUNK,), jnp.float32) for _ in range(3)]
        + [pltpu.VMEM((CHUNK, D), jnp.float32) for _ in range(3)]
        + [pltpu.SemaphoreType.DMA for _ in range(12)]
        + [pltpu.VMEM_SHARED((NP, D), jnp.float32)]
    ),
)(_spmm_body)


def _mlp_body(f_ref, p0_ref, p1_ref, w1t_ref, b1_ref, w2t_ref, b2_ref, o_ref):
    x = f_ref[...] + p0_ref[...] + p1_ref[...]
    h = jnp.maximum(
        jnp.dot(x, w1t_ref[...], preferred_element_type=jnp.float32)
        + b1_ref[...], 0.0)
    o_ref[...] = (jnp.dot(h, w2t_ref[...], preferred_element_type=jnp.float32)
                  + b2_ref[...])


BLK = 1000


def _mlp(feats, p0, p1, w1t, b1, w2t, b2):
    grid = (N // BLK,)
    row_spec = pl.BlockSpec((BLK, D), lambda i: (i, 0))
    full_spec = pl.BlockSpec((D, D), lambda i: (0, 0))
    bias_spec = pl.BlockSpec((1, D), lambda i: (0, 0))
    return pl.pallas_call(
        _mlp_body,
        grid=grid,
        in_specs=[row_spec, row_spec, row_spec,
                  full_spec, bias_spec, full_spec, bias_spec],
        out_specs=row_spec,
        out_shape=jax.ShapeDtypeStruct((N, D), jnp.float32),
    )(feats, p0, p1, w1t, b1, w2t, b2)


def kernel(adj_indices, adj_values, features, W1, b1, W2, b2):
    dst = adj_indices[0]
    src = adj_indices[1]
    partials = _spmm(src, dst, adj_values, features)
    return _mlp(features, partials[0], partials[1],
                W1.T, b1.reshape(1, D), W2.T, b2.reshape(1, D))


# speedup vs baseline: 1.0022x; 1.0022x over previous
import functools

import jax
import jax.numpy as jnp
from jax import lax
from jax.experimental import pallas as pl
from jax.experimental.pallas import tpu as pltpu
from jax.experimental.pallas import tpu_sc as plsc

N = 10000
E = 320000
D = 128
L = 16
NC, NS = 2, 16
NW = NC * NS
NP = 10112

E_PER_W = E // NW
CHUNK = 80
N_CHUNKS = E_PER_W // CHUNK
ROWS_PER_TILE = NP // NS
DEPTH = 3


def _spmm_body(src_hbm, dst_hbm, vals_hbm, feat_hbm, out_hbm,
               src_all, dst0, dst1, dst2, vals0, vals1, vals2,
               rows0, rows1, rows2,
               dsem0, dsem1, dsem2, vsem0, vsem1, vsem2,
               gsem0, gsem1, gsem2, ssem0, ssem1, ssem2, acc_sh):
    cid = lax.axis_index("c")
    sid = lax.axis_index("s")
    wid = sid * NC + cid

    zvec = jnp.zeros((L,), jnp.float32)

    def zblk(e, _):
        for k in range(D // L):
            rows0[e, pl.ds(k * L, L)] = zvec
        return 0

    lax.fori_loop(0, CHUNK, zblk, 0)
    row0 = sid * ROWS_PER_TILE
    for i in range(7):
        pltpu.sync_copy(rows0, acc_sh.at[pl.ds(row0 + i * CHUNK, CHUNK)])
    pltpu.sync_copy(rows0.at[pl.ds(0, ROWS_PER_TILE - 7 * CHUNK)],
                    acc_sh.at[pl.ds(row0 + 7 * CHUNK,
                                    ROWS_PER_TILE - 7 * CHUNK)])
    e0 = wid * E_PER_W
    pltpu.sync_copy(src_hbm.at[pl.ds(e0, E_PER_W)], src_all)
    plsc.subcore_barrier()

    rows = (rows0, rows1, rows2)
    dst = (dst0, dst1, dst2)
    vals = (vals0, vals1, vals2)
    dsem = (dsem0, dsem1, dsem2)
    vsem = (vsem0, vsem1, vsem2)
    gsem = (gsem0, gsem1, gsem2)
    ssem = (ssem0, ssem1, ssem2)

    def scale(s, buf):
        def blk(b, _):
            ve = vals[s][pl.ds(b * L, L)]
            for j in range(L):
                e = b * L + j
                vj = jnp.full((L,), ve[j], dtype=jnp.float32)
                for k in range(D // L):
                    buf[e, pl.ds(k * L, L)] = buf[e, pl.ds(k * L, L)] * vj
            return 0
        lax.fori_loop(0, CHUNK // L, blk, 0)

    def issue_chunk(g, s):
        pltpu.async_copy(dst_hbm.at[pl.ds(e0 + g * CHUNK, CHUNK)],
                         dst[s], dsem[s])
        pltpu.async_copy(vals_hbm.at[pl.ds(e0 + g * CHUNK, CHUNK)],
                         vals[s], vsem[s])
        pltpu.async_copy(feat_hbm.at[src_all.at[pl.ds(g * CHUNK, CHUNK)]],
                         rows[s], gsem[s])

    issue_chunk(0, 0)
    issue_chunk(1, 1)

    def step(g, s, s2, last):
        pltpu.make_async_copy(feat_hbm.at[src_all.at[pl.ds(0, CHUNK)]],
                              rows[s], gsem[s]).wait()

        @pl.when(g >= 1)
        def _():
            pltpu.make_async_copy(rows[s2], acc_sh.at[dst[s2]],
                                  ssem[s2]).wait()

        if not last:
            @pl.when(g + 2 <= N_CHUNKS - 1)
            def _():
                issue_chunk(g + 2, s2)

        pltpu.make_async_copy(vals_hbm.at[pl.ds(e0, CHUNK)], vals[s],
                              vsem[s]).wait()
        scale(s, rows[s])
        pltpu.make_async_copy(dst_hbm.at[pl.ds(e0, CHUNK)], dst[s],
                              dsem[s]).wait()
        if last:
            pltpu.sync_copy(rows[s], acc_sh.at[dst[s]], add=True)
        else:
            pltpu.async_copy(rows[s], acc_sh.at[dst[s]], ssem[s], add=True)

    def outer(i, _):
        for b in range(DEPTH):
            step(i * DEPTH + b, b, (b + 2) % DEPTH, False)
        return 0

    lax.fori_loop(0, (N_CHUNKS - 2) // DEPTH, outer, 0)
    step(N_CHUNKS - 2, 0, 2, False)
    step(N_CHUNKS - 1, 1, 0, True)

    plsc.subcore_barrier()
    pltpu.sync_copy(acc_sh.at[pl.ds(row0, ROWS_PER_TILE)],
                    out_hbm.at[cid, pl.ds(row0, ROWS_PER_TILE)])


_spmm = functools.partial(
    pl.kernel,
    out_type=jax.ShapeDtypeStruct((NC, NP, D), jnp.float32),
    mesh=plsc.VectorSubcoreMesh(core_axis_name="c", subcore_axis_name="s",
                                num_cores=NC, num_subcores=NS),
    scratch_types=(
        [pltpu.VMEM((E_PER_W,), jnp.int32)]
        + [pltpu.VMEM((CHUNK,), jnp.int32) for _ in range(3)]
        + [pltpu.VMEM((CHUNK,), jnp.float32) for _ in range(3)]
        + [pltpu.VMEM((CHUNK, D), jnp.float32) for _ in range(3)]
        + [pltpu.SemaphoreType.DMA for _ in range(12)]
        + [pltpu.VMEM_SHARED((NP, D), jnp.float32)]
    ),
)(_spmm_body)


def _mlp_body(f_ref, p0_ref, p1_ref, w1t_ref, b1_ref, w2t_ref, b2_ref, o_ref):
    x = f_ref[...] + p0_ref[...] + p1_ref[...]
    h = jnp.maximum(
        jnp.dot(x, w1t_ref[...], preferred_element_type=jnp.float32)
        + b1_ref[...], 0.0)
    o_ref[...] = (jnp.dot(h, w2t_ref[...], preferred_element_type=jnp.float32)
                  + b2_ref[...])


BLK = 1000


def _mlp(feats, p0, p1, w1t, b1, w2t, b2):
    grid = (N // BLK,)
    row_spec = pl.BlockSpec((BLK, D), lambda i: (i, 0))
    full_spec = pl.BlockSpec((D, D), lambda i: (0, 0))
    bias_spec = pl.BlockSpec((1, D), lambda i: (0, 0))
    return pl.pallas_call(
        _mlp_body,
        grid=grid,
        in_specs=[row_spec, row_spec, row_spec,
                  full_spec, bias_spec, full_spec, bias_spec],
        out_specs=row_spec,
        out_shape=jax.ShapeDtypeStruct((N, D), jnp.float32),
    )(feats, p0, p1, w1t, b1, w2t, b2)


def kernel(adj_indices, adj_values, features, W1, b1, W2, b2):
    dst = adj_indices[0]
    src = adj_indices[1]
    partials = _spmm(src, dst, adj_values, features)
    return _mlp(features, partials[0], partials[1],
                W1.T, b1.reshape(1, D), W2.T, b2.reshape(1, D))
